# Initial kernel scaffold; baseline (speedup 1.0000x reference)
import jax
import jax.numpy as jnp
from jax.experimental import pallas as pl


def kernel(x, edge_index, W0, as0, ad0, b0, W1, as1, ad1, b1, W2, as2, ad2, b2, W3, as3, ad3, b3, W4, as4, ad4, b4):
    raise NotImplementedError("write your pallas kernel here")


# speedup vs baseline: 5124.8282x; 5124.8282x over previous
import jax
import jax.numpy as jnp
from jax.experimental import pallas as pl
from jax.experimental.pallas import tpu as pltpu

_N = 1024
_BLK = 256


def _gat_body(x_ref, xb_ref, e_ref, w_ref, as_ref, ad_ref, b_ref, out_ref):
    h = jnp.dot(x_ref[...], w_ref[...], preferred_element_type=jnp.float32)
    hb = jnp.dot(xb_ref[...], w_ref[...], preferred_element_type=jnp.float32)
    als = jax.lax.dot_general(
        h, as_ref[...], (((1,), (1,)), ((), ())),
        preferred_element_type=jnp.float32)
    ald = jax.lax.dot_general(
        ad_ref[...], hb, (((1,), (1,)), ((), ())),
        preferred_element_type=jnp.float32)
    s = als + ald
    s = jnp.where(s >= 0.0, s, 0.2 * s)
    mask = e_ref[...] != 0
    s = jnp.where(mask, s, -jnp.inf)
    m = jnp.max(s, axis=0, keepdims=True)
    m = jnp.where(m == -jnp.inf, 0.0, m)
    ex = jnp.where(mask, jnp.exp(s - m), 0.0)
    den = jnp.sum(ex, axis=0, keepdims=True)
    coef = ex / (den + 1e-16)
    out = jax.lax.dot_general(coef, h, (((0,), (0,)), ((), ())),
                              preferred_element_type=jnp.float32)
    out_ref[...] = jnp.maximum(out + b_ref[...], 0.0)


def _gat(x, edge, W, a_s, a_d, b):
    n, din = x.shape
    dout = W.shape[1]
    return pl.pallas_call(
        _gat_body,
        grid=(n // _BLK,),
        in_specs=[
            pl.BlockSpec((n, din), lambda j: (0, 0)),
            pl.BlockSpec((_BLK, din), lambda j: (j, 0)),
            pl.BlockSpec((n, _BLK), lambda j: (0, j)),
            pl.BlockSpec((din, dout), lambda j: (0, 0)),
            pl.BlockSpec((1, dout), lambda j: (0, 0)),
            pl.BlockSpec((1, dout), lambda j: (0, 0)),
            pl.BlockSpec((1, dout), lambda j: (0, 0)),
        ],
        out_specs=pl.BlockSpec((_BLK, dout), lambda j: (j, 0)),
        out_shape=jax.ShapeDtypeStruct((n, dout), jnp.float32),
    )(x, x, edge, W, a_s.reshape(1, -1), a_d.reshape(1, -1), b.reshape(1, -1))


def _recon_body(re_blk_ref, re_ref, out_ref):
    out = jax.lax.dot_general(
        re_blk_ref[...], re_ref[...], (((1,), (1,)), ((), ())),
        preferred_element_type=jnp.float32)
    out_ref[...] = jax.nn.sigmoid(out)


def _recon(re):
    n, d = re.shape
    return pl.pallas_call(
        _recon_body,
        grid=(n // _BLK,),
        in_specs=[
            pl.BlockSpec((_BLK, d), lambda j: (j, 0)),
            pl.BlockSpec((n, d), lambda j: (0, 0)),
        ],
        out_specs=pl.BlockSpec((_BLK, n), lambda j: (j, 0)),
        out_shape=jax.ShapeDtypeStruct((n, n), jnp.float32),
    )(re, re)


def kernel(x, edge_index, W0, as0, ad0, b0, W1, as1, ad1, b1, W2, as2, ad2, b2,
           W3, as3, ad3, b3, W4, as4, ad4, b4):
    e = edge_index
    h = _gat(x, e, W0, as0, ad0, b0)
    z = _gat(h, e, W1, as1, ad1, b1)
    re = _gat(z, e, W2, as2, ad2, b2)
    recon_edge = _recon(re)
    xr = _gat(z, e, W3, as3, ad3, b3)
    xr = _gat(xr, e, W4, as4, ad4, b4)
    return recon_edge, xr, z


# speedup vs baseline: 9495.9017x; 1.8529x over previous
import jax
import jax.numpy as jnp
from jax.experimental import pallas as pl
from jax.experimental.pallas import tpu as pltpu

_N = 1024


def _gat(x, addm, w_ref, as_ref, ad_ref, b_ref):
    h = jnp.dot(x, w_ref[...], preferred_element_type=jnp.float32)
    als = jax.lax.dot_general(
        h, as_ref[...], (((1,), (1,)), ((), ())),
        preferred_element_type=jnp.float32)
    ald = jax.lax.dot_general(
        ad_ref[...], h, (((1,), (1,)), ((), ())),
        preferred_element_type=jnp.float32)
    s = als + ald
    s = jnp.where(s >= 0.0, s, 0.2 * s)
    s = s + addm
    m = jnp.max(s, axis=0, keepdims=True)
    m = jnp.where(m == -jnp.inf, 0.0, m)
    ex = jnp.exp(s - m)
    den = jnp.sum(ex, axis=0, keepdims=True)
    coef = ex / (den + 1e-16)
    out = jax.lax.dot_general(coef, h, (((0,), (0,)), ((), ())),
                              preferred_element_type=jnp.float32)
    return jnp.maximum(out + b_ref[...], 0.0)


def _body(x_ref, e_ref,
          w0, as0, ad0, b0, w1, as1, ad1, b1, w2, as2, ad2, b2,
          w3, as3, ad3, b3, w4, as4, ad4, b4,
          recon_ref, xr_ref, z_ref):
    addm = jnp.where(e_ref[...] != 0, 0.0, -jnp.inf)
    h = _gat(x_ref[...], addm, w0, as0, ad0, b0)
    z = _gat(h, addm, w1, as1, ad1, b1)
    z_ref[...] = z
    re = _gat(z, addm, w2, as2, ad2, b2)
    recon_ref[...] = jax.nn.sigmoid(
        jax.lax.dot_general(re, re, (((1,), (1,)), ((), ())),
                            preferred_element_type=jnp.float32))
    xr = _gat(z, addm, w3, as3, ad3, b3)
    xr_ref[...] = _gat(xr, addm, w4, as4, ad4, b4)


def kernel(x, edge_index, W0, as0, ad0, b0, W1, as1, ad1, b1, W2, as2, ad2, b2,
           W3, as3, ad3, b3, W4, as4, ad4, b4):
    n, in_ch = x.shape
    mid = W1.shape[1]
    recon, xr, z = pl.pallas_call(
        _body,
        out_shape=(
            jax.ShapeDtypeStruct((n, n), jnp.float32),
            jax.ShapeDtypeStruct((n, in_ch), jnp.float32),
            jax.ShapeDtypeStruct((n, mid), jnp.float32),
        ),
    )(x, edge_index,
      W0, as0.reshape(1, -1), ad0.reshape(1, -1), b0.reshape(1, -1),
      W1, as1.reshape(1, -1), ad1.reshape(1, -1), b1.reshape(1, -1),
      W2, as2.reshape(1, -1), ad2.reshape(1, -1), b2.reshape(1, -1),
      W3, as3.reshape(1, -1), ad3.reshape(1, -1), b3.reshape(1, -1),
      W4, as4.reshape(1, -1), ad4.reshape(1, -1), b4.reshape(1, -1))
    return recon, xr, z


# speedup vs baseline: 10311.7538x; 1.0859x over previous
import jax
import jax.numpy as jnp
from jax.experimental import pallas as pl
from jax.experimental.pallas import tpu as pltpu

_N = 1024


def _gat(x, addm, w_ref, as_ref, ad_ref, b_ref):
    h = jnp.dot(x, w_ref[...], preferred_element_type=jnp.float32)
    als = jax.lax.dot_general(
        h, as_ref[...], (((1,), (1,)), ((), ())),
        preferred_element_type=jnp.float32)
    ald = jax.lax.dot_general(
        ad_ref[...], h, (((1,), (1,)), ((), ())),
        preferred_element_type=jnp.float32)
    s = als + ald
    s = jnp.maximum(s, 0.2 * s)
    s = s + addm
    m = jnp.max(s, axis=0, keepdims=True)
    m = jnp.where(m == -jnp.inf, 0.0, m)
    ex = jnp.exp(s - m)
    den = jnp.sum(ex, axis=0, keepdims=True)
    out = jax.lax.dot_general(ex, h, (((0,), (0,)), ((), ())),
                              preferred_element_type=jnp.float32)
    out = out / (den.reshape(-1, 1) + 1e-16)
    return jnp.maximum(out + b_ref[...], 0.0)


def _body(x_ref, e_ref,
          w0, as0, ad0, b0, w1, as1, ad1, b1, w2, as2, ad2, b2,
          w3, as3, ad3, b3, w4, as4, ad4, b4,
          recon_ref, xr_ref, z_ref):
    addm = jnp.where(e_ref[...] != 0, 0.0, -jnp.inf)
    h = _gat(x_ref[...], addm, w0, as0, ad0, b0)
    z = _gat(h, addm, w1, as1, ad1, b1)
    z_ref[...] = z
    re = _gat(z, addm, w2, as2, ad2, b2)
    recon_ref[...] = jax.nn.sigmoid(
        jax.lax.dot_general(re, re, (((1,), (1,)), ((), ())),
                            preferred_element_type=jnp.float32))
    xr = _gat(z, addm, w3, as3, ad3, b3)
    xr_ref[...] = _gat(xr, addm, w4, as4, ad4, b4)


def kernel(x, edge_index, W0, as0, ad0, b0, W1, as1, ad1, b1, W2, as2, ad2, b2,
           W3, as3, ad3, b3, W4, as4, ad4, b4):
    n, in_ch = x.shape
    mid = W1.shape[1]
    recon, xr, z = pl.pallas_call(
        _body,
        out_shape=(
            jax.ShapeDtypeStruct((n, n), jnp.float32),
            jax.ShapeDtypeStruct((n, in_ch), jnp.float32),
            jax.ShapeDtypeStruct((n, mid), jnp.float32),
        ),
    )(x, edge_index,
      W0, as0.reshape(1, -1), ad0.reshape(1, -1), b0.reshape(1, -1),
      W1, as1.reshape(1, -1), ad1.reshape(1, -1), b1.reshape(1, -1),
      W2, as2.reshape(1, -1), ad2.reshape(1, -1), b2.reshape(1, -1),
      W3, as3.reshape(1, -1), ad3.reshape(1, -1), b3.reshape(1, -1),
      W4, as4.reshape(1, -1), ad4.reshape(1, -1), b4.reshape(1, -1))
    return recon, xr, z
